# Initial kernel scaffold; baseline (speedup 1.0000x reference)
import jax
import jax.numpy as jnp
from jax.experimental import pallas as pl


def kernel(x, gate_w, expert_w1, expert_w2, ln_gamma, ln_beta):
    raise NotImplementedError("write your pallas kernel here")


# speedup vs baseline: 4.7724x; 4.7724x over previous
import functools

import jax
import jax.numpy as jnp
from jax import lax
from jax.experimental import pallas as pl
from jax.experimental.pallas import tpu as pltpu
from jax.experimental.pallas import tpu_sc as plsc

B_, T_, D_, H_, E_ = 2, 2048, 1024, 2048, 8
N_ = B_ * T_
BLK = 256
NBLK = (N_ + E_ * (BLK - 1) + BLK - 1) // BLK
NPAD = NBLK * BLK
CHUNK = 1024
NH = 2
HC = H_ // NH

SC_NC = 2
SC_NS = 16
SC_W = SC_NC * SC_NS
TOK_PER_W = N_ // SC_W
SC_CH = 64
SC_NCH = TOK_PER_W // SC_CH


def _routing_body(x_ref, gw_ref, pos_ref, be_ref):
    xv = x_ref[...]
    gw = gw_ref[...]
    logits = lax.dot_general(xv, gw, (((1,), (1,)), ((), ())),
                             preferred_element_type=jnp.float32)
    maxl = jnp.max(logits, axis=1, keepdims=True)
    p = jnp.exp(logits - maxl)
    p = p / jnp.sum(p, axis=1, keepdims=True)
    maxp = jnp.max(p, axis=1, keepdims=True)
    eids = lax.broadcasted_iota(jnp.int32, (N_, E_), 1)
    cand = jnp.where(p >= maxp, eids, E_)
    eidx = jnp.min(cand, axis=1, keepdims=True)
    onehot = (eids == eidx).astype(jnp.float32)

    r_i = lax.broadcasted_iota(jnp.int32, (CHUNK, CHUNK), 0)
    c_i = lax.broadcasted_iota(jnp.int32, (CHUNK, CHUNK), 1)
    tril = (r_i >= c_i).astype(jnp.float32)
    base = jnp.zeros((1, E_), jnp.float32)
    cum_rows = []
    for c in range(N_ // CHUNK):
        oc = lax.slice(onehot, (c * CHUNK, 0), ((c + 1) * CHUNK, E_))
        local = jnp.dot(tril, oc, preferred_element_type=jnp.float32)
        cum_rows.append(local + base)
        base = base + jnp.sum(oc, axis=0, keepdims=True)
    cum = jnp.concatenate(cum_rows, axis=0)
    counts = base

    pc = jnp.ceil(counts / BLK) * BLK
    er = lax.broadcasted_iota(jnp.int32, (E_, E_), 0)
    ec = lax.broadcasted_iota(jnp.int32, (E_, E_), 1)
    sut = (er < ec).astype(jnp.float32)
    off8 = jnp.dot(jnp.broadcast_to(pc, (E_, E_)), sut,
                   preferred_element_type=jnp.float32)
    off = lax.slice(off8, (0, 0), (1, E_))

    rank = jnp.sum(cum * onehot, axis=1, keepdims=True) - 1.0
    poff = jnp.sum(off * onehot, axis=1, keepdims=True)
    pos = (poff + rank).astype(jnp.int32)
    pos_ref[...] = jnp.broadcast_to(pos, (N_, E_))

    bstart = (lax.broadcasted_iota(jnp.int32, (NBLK, E_), 0) * BLK
              ).astype(jnp.float32)
    beids = lax.broadcasted_iota(jnp.int32, (NBLK, E_), 1)
    offb = jnp.broadcast_to(off, (NBLK, E_))
    pcb = jnp.broadcast_to(pc, (NBLK, E_))
    ok = (offb <= bstart) & (pcb > 0)
    be = jnp.max(jnp.where(ok, beids, 0), axis=1, keepdims=True)
    be_ref[...] = jnp.broadcast_to(be, (NBLK, E_))


_routing = pl.pallas_call(
    _routing_body,
    out_shape=(
        jax.ShapeDtypeStruct((N_, E_), jnp.int32),
        jax.ShapeDtypeStruct((NBLK, E_), jnp.int32),
    ),
)


def _mlp_body(be_ref, xs_ref, w1_ref, w2_ref, g_ref, b_ref, out_ref, acc_ref):
    h = pl.program_id(1)
    hpre = jnp.dot(xs_ref[...], w1_ref[0],
                   preferred_element_type=jnp.float32)
    hact = 0.5 * hpre * (1.0 + lax.erf(hpre * 0.7071067811865476))
    part = jnp.dot(hact, w2_ref[0],
                   preferred_element_type=jnp.float32)

    @pl.when(h == 0)
    def _():
        acc_ref[...] = part

    @pl.when(h != 0)
    def _():
        acc_ref[...] += part

    @pl.when(h == NH - 1)
    def _():
        y = acc_ref[...]
        mean = jnp.mean(y, axis=1, keepdims=True)
        yc = y - mean
        var = jnp.mean(yc * yc, axis=1, keepdims=True)
        out_ref[...] = yc * lax.rsqrt(var + 1e-5) * g_ref[...] + b_ref[...]


_mlp = pl.pallas_call(
    _mlp_body,
    grid_spec=pltpu.PrefetchScalarGridSpec(
        num_scalar_prefetch=1,
        grid=(NBLK, NH),
        in_specs=[
            pl.BlockSpec((BLK, D_), lambda b, h, be: (b, 0)),
            pl.BlockSpec((1, D_, HC), lambda b, h, be: (be[b], 0, h)),
            pl.BlockSpec((1, HC, D_), lambda b, h, be: (be[b], h, 0)),
            pl.BlockSpec((1, D_), lambda b, h, be: (0, 0)),
            pl.BlockSpec((1, D_), lambda b, h, be: (0, 0)),
        ],
        out_specs=pl.BlockSpec((BLK, D_), lambda b, h, be: (b, 0)),
        scratch_shapes=[pltpu.VMEM((BLK, D_), jnp.float32)],
    ),
    out_shape=jax.ShapeDtypeStruct((NPAD, D_), jnp.float32),
    compiler_params=pltpu.CompilerParams(
        dimension_semantics=("arbitrary", "arbitrary"),
    ),
)


@functools.cache
def _sc_kernels():
    mesh = plsc.VectorSubcoreMesh(core_axis_name="c", subcore_axis_name="s")
    scratch = [
        pltpu.VMEM((SC_CH,), jnp.int32),
        pltpu.VMEM((SC_CH, D_), jnp.float32),
        pltpu.SemaphoreType.DMA,
    ]

    @functools.partial(
        pl.kernel,
        mesh=mesh,
        out_type=jax.ShapeDtypeStruct((NPAD, D_), jnp.float32),
        scratch_types=scratch,
    )
    def sc_scatter(x_hbm, pos_hbm, xs_hbm, idx_v, rows_v, sem):
        wid = lax.axis_index("s") * SC_NC + lax.axis_index("c")
        for ch in range(SC_NCH):
            base = wid * TOK_PER_W + ch * SC_CH
            pltpu.sync_copy(pos_hbm.at[pl.ds(base, SC_CH)], idx_v)
            pltpu.sync_copy(x_hbm.at[pl.ds(base, SC_CH)], rows_v)
            pltpu.async_copy(rows_v, xs_hbm.at[idx_v], sem).wait()

    @functools.partial(
        pl.kernel,
        mesh=mesh,
        out_type=jax.ShapeDtypeStruct((N_, D_), jnp.float32),
        scratch_types=scratch,
    )
    def sc_gather(ys_hbm, pos_hbm, out_hbm, idx_v, rows_v, sem):
        wid = lax.axis_index("s") * SC_NC + lax.axis_index("c")
        for ch in range(SC_NCH):
            base = wid * TOK_PER_W + ch * SC_CH
            pltpu.sync_copy(pos_hbm.at[pl.ds(base, SC_CH)], idx_v)
            pltpu.async_copy(ys_hbm.at[idx_v], rows_v, sem).wait()
            pltpu.sync_copy(rows_v, out_hbm.at[pl.ds(base, SC_CH)])

    return sc_scatter, sc_gather


def kernel(x, gate_w, expert_w1, expert_w2, ln_gamma, ln_beta):
    x_flat = x.reshape(N_, D_)
    pos8, be8 = _routing(x_flat, gate_w)
    pos = pos8[:, 0]
    be = be8[:, 0]
    sc_scatter, sc_gather = _sc_kernels()
    xs = sc_scatter(x_flat, pos)
    ys = _mlp(be, xs, expert_w1, expert_w2,
              ln_gamma.reshape(1, D_), ln_beta.reshape(1, D_))
    out = sc_gather(ys, pos)
    return out.reshape(B_, T_, D_)


# speedup vs baseline: 4.7774x; 1.0010x over previous
import functools

import jax
import jax.numpy as jnp
from jax import lax
from jax.experimental import pallas as pl
from jax.experimental.pallas import tpu as pltpu
from jax.experimental.pallas import tpu_sc as plsc

B_, T_, D_, H_, E_ = 2, 2048, 1024, 2048, 8
N_ = B_ * T_
BLK = 256
NBLK = (N_ + E_ * (BLK - 1) + BLK - 1) // BLK
NPAD = NBLK * BLK
CHUNK = 1024
NH = 2
HC = H_ // NH
NMETA = NBLK + 8

SC_NC = 2
SC_NS = 16
SC_W = SC_NC * SC_NS
TOK_PER_W = N_ // SC_W
SC_CH = 64
SC_NCH = TOK_PER_W // SC_CH


def _routing_body(x_ref, gw_ref, pos_ref, be_ref):
    xv = x_ref[...]
    gw = gw_ref[...]
    logits = lax.dot_general(xv, gw, (((1,), (1,)), ((), ())),
                             preferred_element_type=jnp.float32)
    maxl = jnp.max(logits, axis=1, keepdims=True)
    p = jnp.exp(logits - maxl)
    p = p / jnp.sum(p, axis=1, keepdims=True)
    maxp = jnp.max(p, axis=1, keepdims=True)
    eids = lax.broadcasted_iota(jnp.int32, (N_, E_), 1)
    cand = jnp.where(p >= maxp, eids, E_)
    eidx = jnp.min(cand, axis=1, keepdims=True)
    onehot = (eids == eidx).astype(jnp.float32)

    r_i = lax.broadcasted_iota(jnp.int32, (CHUNK, CHUNK), 0)
    c_i = lax.broadcasted_iota(jnp.int32, (CHUNK, CHUNK), 1)
    tril = (r_i >= c_i).astype(jnp.bfloat16)
    base = jnp.zeros((1, E_), jnp.float32)
    cum_rows = []
    for c in range(N_ // CHUNK):
        oc = lax.slice(onehot, (c * CHUNK, 0), ((c + 1) * CHUNK, E_))
        local = jnp.dot(tril, oc.astype(jnp.bfloat16),
                        preferred_element_type=jnp.float32)
        cum_rows.append(local + base)
        base = base + jnp.sum(oc, axis=0, keepdims=True)
    cum = jnp.concatenate(cum_rows, axis=0)
    counts = base

    pc = jnp.ceil(counts / BLK) * BLK
    er = lax.broadcasted_iota(jnp.int32, (E_, E_), 0)
    ec = lax.broadcasted_iota(jnp.int32, (E_, E_), 1)
    sut = (er < ec).astype(jnp.float32)
    off8 = jnp.dot(jnp.broadcast_to(pc, (E_, E_)), sut,
                   preferred_element_type=jnp.float32)
    off = lax.slice(off8, (0, 0), (1, E_))

    rank = jnp.sum(cum * onehot, axis=1, keepdims=True) - 1.0
    poff = jnp.sum(off * onehot, axis=1, keepdims=True)
    pos = (poff + rank).astype(jnp.int32)
    pos_ref[...] = jnp.broadcast_to(pos, (N_, E_))

    bstart = (lax.broadcasted_iota(jnp.int32, (NBLK, E_), 0) * BLK
              ).astype(jnp.float32)
    beids = lax.broadcasted_iota(jnp.int32, (NBLK, E_), 1)
    offb = jnp.broadcast_to(off, (NBLK, E_))
    pcb = jnp.broadcast_to(pc, (NBLK, E_))
    ok = (offb <= bstart) & (pcb > 0)
    be = jnp.max(jnp.where(ok, beids, 0), axis=1, keepdims=True)
    nreal = (jnp.sum(pc, axis=1, keepdims=True) / BLK).astype(jnp.int32)
    meta = jnp.concatenate([be, jnp.broadcast_to(nreal, (NMETA - NBLK, 1))],
                           axis=0)
    be_ref[...] = jnp.broadcast_to(meta, (NMETA, E_))


_routing = pl.pallas_call(
    _routing_body,
    out_shape=(
        jax.ShapeDtypeStruct((N_, E_), jnp.int32),
        jax.ShapeDtypeStruct((NMETA, E_), jnp.int32),
    ),
)


def _mlp_body(meta_ref, xs_ref, w1_ref, w2_ref, g_ref, b_ref, out_ref,
              acc_ref):
    blk = pl.program_id(0)
    h = pl.program_id(1)

    @pl.when(blk < meta_ref[NBLK])
    def _():
        hpre = jnp.dot(xs_ref[...].astype(jnp.bfloat16),
                       w1_ref[0].astype(jnp.bfloat16),
                       preferred_element_type=jnp.float32)
        hact = 0.5 * hpre * (1.0 + lax.erf(hpre * 0.7071067811865476))
        part = jnp.dot(hact.astype(jnp.bfloat16),
                       w2_ref[0].astype(jnp.bfloat16),
                       preferred_element_type=jnp.float32)

        @pl.when(h == 0)
        def _():
            acc_ref[...] = part

        @pl.when(h != 0)
        def _():
            acc_ref[...] += part

        @pl.when(h == NH - 1)
        def _():
            y = acc_ref[...]
            mean = jnp.mean(y, axis=1, keepdims=True)
            yc = y - mean
            var = jnp.mean(yc * yc, axis=1, keepdims=True)
            out_ref[...] = (yc * lax.rsqrt(var + 1e-5) * g_ref[...]
                            + b_ref[...])


_mlp = pl.pallas_call(
    _mlp_body,
    grid_spec=pltpu.PrefetchScalarGridSpec(
        num_scalar_prefetch=1,
        grid=(NBLK, NH),
        in_specs=[
            pl.BlockSpec((BLK, D_), lambda b, h, be: (b, 0)),
            pl.BlockSpec((1, D_, HC), lambda b, h, be: (be[b], 0, h)),
            pl.BlockSpec((1, HC, D_), lambda b, h, be: (be[b], h, 0)),
            pl.BlockSpec((1, D_), lambda b, h, be: (0, 0)),
            pl.BlockSpec((1, D_), lambda b, h, be: (0, 0)),
        ],
        out_specs=pl.BlockSpec((BLK, D_), lambda b, h, be: (b, 0)),
        scratch_shapes=[pltpu.VMEM((BLK, D_), jnp.float32)],
    ),
    out_shape=jax.ShapeDtypeStruct((NPAD, D_), jnp.float32),
    compiler_params=pltpu.CompilerParams(
        dimension_semantics=("arbitrary", "arbitrary"),
    ),
)


@functools.cache
def _sc_kernels():
    mesh = plsc.VectorSubcoreMesh(core_axis_name="c", subcore_axis_name="s")
    scratch = [
        pltpu.VMEM((SC_CH,), jnp.int32),
        pltpu.VMEM((SC_CH, D_), jnp.float32),
        pltpu.SemaphoreType.DMA,
    ]

    @functools.partial(
        pl.kernel,
        mesh=mesh,
        out_type=jax.ShapeDtypeStruct((NPAD, D_), jnp.float32),
        scratch_types=scratch,
    )
    def sc_scatter(x_hbm, pos_hbm, xs_hbm, idx_v, rows_v, sem):
        wid = lax.axis_index("s") * SC_NC + lax.axis_index("c")
        for ch in range(SC_NCH):
            base = wid * TOK_PER_W + ch * SC_CH
            pltpu.sync_copy(pos_hbm.at[pl.ds(base, SC_CH)], idx_v)
            pltpu.sync_copy(x_hbm.at[pl.ds(base, SC_CH)], rows_v)
            pltpu.async_copy(rows_v, xs_hbm.at[idx_v], sem).wait()

    @functools.partial(
        pl.kernel,
        mesh=mesh,
        out_type=jax.ShapeDtypeStruct((N_, D_), jnp.float32),
        scratch_types=scratch,
    )
    def sc_gather(ys_hbm, pos_hbm, out_hbm, idx_v, rows_v, sem):
        wid = lax.axis_index("s") * SC_NC + lax.axis_index("c")
        for ch in range(SC_NCH):
            base = wid * TOK_PER_W + ch * SC_CH
            pltpu.sync_copy(pos_hbm.at[pl.ds(base, SC_CH)], idx_v)
            pltpu.async_copy(ys_hbm.at[idx_v], rows_v, sem).wait()
            pltpu.sync_copy(rows_v, out_hbm.at[pl.ds(base, SC_CH)])

    return sc_scatter, sc_gather


def kernel(x, gate_w, expert_w1, expert_w2, ln_gamma, ln_beta):
    x_flat = x.reshape(N_, D_)
    pos8, be8 = _routing(x_flat, gate_w)
    pos = pos8[:, 0]
    be = be8[:, 0]
    sc_scatter, sc_gather = _sc_kernels()
    xs = sc_scatter(x_flat, pos)
    ys = _mlp(be, xs, expert_w1, expert_w2,
              ln_gamma.reshape(1, D_), ln_beta.reshape(1, D_))
    out = sc_gather(ys, pos)
    return out.reshape(B_, T_, D_)


# speedup vs baseline: 6.0920x; 1.2752x over previous
import functools

import jax
import jax.numpy as jnp
from jax import lax
from jax.experimental import pallas as pl
from jax.experimental.pallas import tpu as pltpu
from jax.experimental.pallas import tpu_sc as plsc

B_, T_, D_, H_, E_ = 2, 2048, 1024, 2048, 8
N_ = B_ * T_
BLK = 256
NBLK = (N_ + E_ * (BLK - 1) + BLK - 1) // BLK
NPAD = NBLK * BLK
CHUNK = 1024
NMETA = NBLK + 8

SC_NC = 2
SC_NS = 16
SC_W = SC_NC * SC_NS
TOK_PER_W = N_ // SC_W
SC_CH = 64
SC_NCH = TOK_PER_W // SC_CH


def _routing_body(x_ref, gw_ref, pos_ref, be_ref):
    xv = x_ref[...]
    gw = gw_ref[...]
    logits = lax.dot_general(xv, gw, (((1,), (1,)), ((), ())),
                             preferred_element_type=jnp.float32)
    maxl = jnp.max(logits, axis=1, keepdims=True)
    p = jnp.exp(logits - maxl)
    p = p / jnp.sum(p, axis=1, keepdims=True)
    maxp = jnp.max(p, axis=1, keepdims=True)
    eids = lax.broadcasted_iota(jnp.int32, (N_, E_), 1)
    cand = jnp.where(p >= maxp, eids, E_)
    eidx = jnp.min(cand, axis=1, keepdims=True)
    onehot = (eids == eidx).astype(jnp.float32)

    r_i = lax.broadcasted_iota(jnp.int32, (CHUNK, CHUNK), 0)
    c_i = lax.broadcasted_iota(jnp.int32, (CHUNK, CHUNK), 1)
    tril = (r_i >= c_i).astype(jnp.bfloat16)
    base = jnp.zeros((1, E_), jnp.float32)
    cum_rows = []
    for c in range(N_ // CHUNK):
        oc = lax.slice(onehot, (c * CHUNK, 0), ((c + 1) * CHUNK, E_))
        local = jnp.dot(tril, oc.astype(jnp.bfloat16),
                        preferred_element_type=jnp.float32)
        cum_rows.append(local + base)
        base = base + jnp.sum(oc, axis=0, keepdims=True)
    cum = jnp.concatenate(cum_rows, axis=0)
    counts = base

    pc = jnp.ceil(counts / BLK) * BLK
    er = lax.broadcasted_iota(jnp.int32, (E_, E_), 0)
    ec = lax.broadcasted_iota(jnp.int32, (E_, E_), 1)
    sut = (er < ec).astype(jnp.float32)
    off8 = jnp.dot(jnp.broadcast_to(pc, (E_, E_)), sut,
                   preferred_element_type=jnp.float32)
    off = lax.slice(off8, (0, 0), (1, E_))

    rank = jnp.sum(cum * onehot, axis=1, keepdims=True) - 1.0
    poff = jnp.sum(off * onehot, axis=1, keepdims=True)
    pos = (poff + rank).astype(jnp.int32)
    pos_ref[...] = jnp.broadcast_to(pos, (N_, E_))

    bstart = (lax.broadcasted_iota(jnp.int32, (NBLK, E_), 0) * BLK
              ).astype(jnp.float32)
    beids = lax.broadcasted_iota(jnp.int32, (NBLK, E_), 1)
    offb = jnp.broadcast_to(off, (NBLK, E_))
    pcb = jnp.broadcast_to(pc, (NBLK, E_))
    ok = (offb <= bstart) & (pcb > 0)
    be = jnp.max(jnp.where(ok, beids, 0), axis=1, keepdims=True)
    nreal = (jnp.sum(pc, axis=1, keepdims=True) / BLK).astype(jnp.int32)
    meta = jnp.concatenate([be, jnp.broadcast_to(nreal, (NMETA - NBLK, 1))],
                           axis=0)
    be_ref[...] = jnp.broadcast_to(meta, (NMETA, E_))


_routing = pl.pallas_call(
    _routing_body,
    out_shape=(
        jax.ShapeDtypeStruct((N_, E_), jnp.int32),
        jax.ShapeDtypeStruct((NMETA, E_), jnp.int32),
    ),
)


def _mlp_body(meta_ref, xs_ref, w1_ref, w2_ref, g_ref, b_ref, out_ref):
    blk = pl.program_id(0)

    @pl.when(blk < meta_ref[NBLK])
    def _():
        hpre = jnp.dot(xs_ref[...].astype(jnp.bfloat16),
                       w1_ref[0].astype(jnp.bfloat16),
                       preferred_element_type=jnp.float32)
        hact = 0.5 * hpre * (1.0 + lax.erf(hpre * 0.7071067811865476))
        y = jnp.dot(hact.astype(jnp.bfloat16),
                    w2_ref[0].astype(jnp.bfloat16),
                    preferred_element_type=jnp.float32)
        mean = jnp.mean(y, axis=1, keepdims=True)
        yc = y - mean
        var = jnp.mean(yc * yc, axis=1, keepdims=True)
        out_ref[...] = yc * lax.rsqrt(var + 1e-5) * g_ref[...] + b_ref[...]


_mlp = pl.pallas_call(
    _mlp_body,
    grid_spec=pltpu.PrefetchScalarGridSpec(
        num_scalar_prefetch=1,
        grid=(NBLK,),
        in_specs=[
            pl.BlockSpec((BLK, D_), lambda b, be: (b, 0)),
            pl.BlockSpec((1, D_, H_), lambda b, be: (be[b], 0, 0)),
            pl.BlockSpec((1, H_, D_), lambda b, be: (be[b], 0, 0)),
            pl.BlockSpec((1, D_), lambda b, be: (0, 0)),
            pl.BlockSpec((1, D_), lambda b, be: (0, 0)),
        ],
        out_specs=pl.BlockSpec((BLK, D_), lambda b, be: (b, 0)),
    ),
    out_shape=jax.ShapeDtypeStruct((NPAD, D_), jnp.float32),
    compiler_params=pltpu.CompilerParams(
        dimension_semantics=("arbitrary",),
    ),
)


@functools.cache
def _sc_kernels():
    mesh = plsc.VectorSubcoreMesh(core_axis_name="c", subcore_axis_name="s")
    scratch = [
        pltpu.VMEM((SC_CH,), jnp.int32),
        pltpu.VMEM((SC_CH, D_), jnp.float32),
        pltpu.SemaphoreType.DMA,
    ]

    @functools.partial(
        pl.kernel,
        mesh=mesh,
        out_type=jax.ShapeDtypeStruct((NPAD, D_), jnp.float32),
        scratch_types=scratch,
    )
    def sc_scatter(x_hbm, pos_hbm, xs_hbm, idx_v, rows_v, sem):
        wid = lax.axis_index("s") * SC_NC + lax.axis_index("c")
        for ch in range(SC_NCH):
            base = wid * TOK_PER_W + ch * SC_CH
            pltpu.sync_copy(pos_hbm.at[pl.ds(base, SC_CH)], idx_v)
            pltpu.sync_copy(x_hbm.at[pl.ds(base, SC_CH)], rows_v)
            pltpu.async_copy(rows_v, xs_hbm.at[idx_v], sem).wait()

    @functools.partial(
        pl.kernel,
        mesh=mesh,
        out_type=jax.ShapeDtypeStruct((N_, D_), jnp.float32),
        scratch_types=scratch,
    )
    def sc_gather(ys_hbm, pos_hbm, out_hbm, idx_v, rows_v, sem):
        wid = lax.axis_index("s") * SC_NC + lax.axis_index("c")
        for ch in range(SC_NCH):
            base = wid * TOK_PER_W + ch * SC_CH
            pltpu.sync_copy(pos_hbm.at[pl.ds(base, SC_CH)], idx_v)
            pltpu.async_copy(ys_hbm.at[idx_v], rows_v, sem).wait()
            pltpu.sync_copy(rows_v, out_hbm.at[pl.ds(base, SC_CH)])

    return sc_scatter, sc_gather


def kernel(x, gate_w, expert_w1, expert_w2, ln_gamma, ln_beta):
    x_flat = x.reshape(N_, D_)
    pos8, be8 = _routing(x_flat, gate_w)
    pos = pos8[:, 0]
    be = be8[:, 0]
    sc_scatter, sc_gather = _sc_kernels()
    xs = sc_scatter(x_flat, pos)
    ys = _mlp(be, xs, expert_w1, expert_w2,
              ln_gamma.reshape(1, D_), ln_beta.reshape(1, D_))
    out = sc_gather(ys, pos)
    return out.reshape(B_, T_, D_)


# speedup vs baseline: 7.1108x; 1.1672x over previous
import functools

import jax
import jax.numpy as jnp
from jax import lax
from jax.experimental import pallas as pl
from jax.experimental.pallas import tpu as pltpu
from jax.experimental.pallas import tpu_sc as plsc

B_, T_, D_, H_, E_ = 2, 2048, 1024, 2048, 8
N_ = B_ * T_
BLK = 256
NBLK = (N_ + E_ * (BLK - 1) + BLK - 1) // BLK
NPAD = NBLK * BLK
CHUNK = 1024
OFF_SLOT = NBLK
OFF_BND = 2 * NBLK
OFF_NXE = 3 * NBLK
OFF_NREAL = 4 * NBLK
NMETA = 4 * NBLK + 8

SC_NC = 2
SC_NS = 16
SC_W = SC_NC * SC_NS
TOK_PER_W = N_ // SC_W
SC_CH = 64
SC_NCH = TOK_PER_W // SC_CH


def _routing_body(x_ref, gw_ref, pos_ref, be_ref):
    xv = x_ref[...]
    gw = gw_ref[...]
    logits = lax.dot_general(xv, gw, (((1,), (1,)), ((), ())),
                             preferred_element_type=jnp.float32)
    maxl = jnp.max(logits, axis=1, keepdims=True)
    p = jnp.exp(logits - maxl)
    p = p / jnp.sum(p, axis=1, keepdims=True)
    maxp = jnp.max(p, axis=1, keepdims=True)
    eids = lax.broadcasted_iota(jnp.int32, (N_, E_), 1)
    cand = jnp.where(p >= maxp, eids, E_)
    eidx = jnp.min(cand, axis=1, keepdims=True)
    onehot = (eids == eidx).astype(jnp.float32)

    r_i = lax.broadcasted_iota(jnp.int32, (CHUNK, CHUNK), 0)
    c_i = lax.broadcasted_iota(jnp.int32, (CHUNK, CHUNK), 1)
    tril = (r_i >= c_i).astype(jnp.bfloat16)
    base = jnp.zeros((1, E_), jnp.float32)
    cum_rows = []
    for c in range(N_ // CHUNK):
        oc = lax.slice(onehot, (c * CHUNK, 0), ((c + 1) * CHUNK, E_))
        local = jnp.dot(tril, oc.astype(jnp.bfloat16),
                        preferred_element_type=jnp.float32)
        cum_rows.append(local + base)
        base = base + jnp.sum(oc, axis=0, keepdims=True)
    cum = jnp.concatenate(cum_rows, axis=0)
    counts = base

    pc = jnp.ceil(counts / BLK) * BLK
    er = lax.broadcasted_iota(jnp.int32, (E_, E_), 0)
    ec = lax.broadcasted_iota(jnp.int32, (E_, E_), 1)
    sut = (er < ec).astype(jnp.float32)
    off8 = jnp.dot(jnp.broadcast_to(pc, (E_, E_)), sut,
                   preferred_element_type=jnp.float32)
    off = lax.slice(off8, (0, 0), (1, E_))

    rank = jnp.sum(cum * onehot, axis=1, keepdims=True) - 1.0
    poff = jnp.sum(off * onehot, axis=1, keepdims=True)
    pos = (poff + rank).astype(jnp.int32)
    pos_ref[...] = jnp.broadcast_to(pos, (N_, E_))

    bstart = (lax.broadcasted_iota(jnp.int32, (NBLK, E_), 0) * BLK
              ).astype(jnp.float32)
    beids = lax.broadcasted_iota(jnp.int32, (NBLK, E_), 1)
    offb = jnp.broadcast_to(off, (NBLK, E_))
    pcb = jnp.broadcast_to(pc, (NBLK, E_))
    ok = (offb <= bstart) & (pcb > 0)
    be = jnp.max(jnp.where(ok, beids, 0), axis=1, keepdims=True)
    beb = jnp.broadcast_to(be, (NBLK, E_))
    okn = (pcb > 0) & (beids > beb)
    nxt = jnp.min(jnp.where(okn, beids, E_), axis=1, keepdims=True)
    nxe = jnp.where(nxt == E_, be, nxt)
    okr = (pcb > 0) & (beids <= beb)
    rid = jnp.sum(okr.astype(jnp.int32), axis=1, keepdims=True) - 1
    slot = jnp.bitwise_and(rid, 1)
    be_prev = jnp.concatenate(
        [jnp.full((1, 1), -1, jnp.int32),
         lax.slice(be, (0, 0), (NBLK - 1, 1))], axis=0)
    bnd = (be != be_prev).astype(jnp.int32)
    nreal = (jnp.sum(pc, axis=1, keepdims=True) / BLK).astype(jnp.int32)
    meta = jnp.concatenate(
        [be, slot, bnd, nxe,
         jnp.broadcast_to(nreal, (NMETA - 4 * NBLK, 1))], axis=0)
    be_ref[...] = jnp.broadcast_to(meta, (NMETA, E_))


_routing = pl.pallas_call(
    _routing_body,
    out_shape=(
        jax.ShapeDtypeStruct((N_, E_), jnp.int32),
        jax.ShapeDtypeStruct((NMETA, E_), jnp.int32),
    ),
)


def _mlp_body(meta_ref, xs_ref, w1_hbm, w2_hbm, g_ref, b_ref, out_ref,
              w1buf, w2buf, sems):
    blk = pl.program_id(0)

    def w_copies(e, s):
        return (pltpu.make_async_copy(w1_hbm.at[e], w1buf.at[s], sems.at[s]),
                pltpu.make_async_copy(w2_hbm.at[e], w2buf.at[s], sems.at[s]))

    @pl.when(blk < meta_ref[OFF_NREAL])
    def _():
        e = meta_ref[blk]
        slot = meta_ref[OFF_SLOT + blk]
        first = meta_ref[OFF_BND + blk]
        nxe = meta_ref[OFF_NXE + blk]

        @pl.when(blk == 0)
        def _():
            for c in w_copies(e, slot):
                c.start()

        @pl.when(first == 1)
        def _():
            for c in w_copies(e, slot):
                c.wait()

            @pl.when(nxe != e)
            def _():
                for c in w_copies(nxe, 1 - slot):
                    c.start()

        hpre = jnp.dot(xs_ref[...].astype(jnp.bfloat16),
                       w1buf[slot].astype(jnp.bfloat16),
                       preferred_element_type=jnp.float32)
        hact = 0.5 * hpre * (1.0 + lax.erf(hpre * 0.7071067811865476))
        y = jnp.dot(hact.astype(jnp.bfloat16),
                    w2buf[slot].astype(jnp.bfloat16),
                    preferred_element_type=jnp.float32)
        mean = jnp.mean(y, axis=1, keepdims=True)
        yc = y - mean
        var = jnp.mean(yc * yc, axis=1, keepdims=True)
        out_ref[...] = yc * lax.rsqrt(var + 1e-5) * g_ref[...] + b_ref[...]


_mlp = pl.pallas_call(
    _mlp_body,
    grid_spec=pltpu.PrefetchScalarGridSpec(
        num_scalar_prefetch=1,
        grid=(NBLK,),
        in_specs=[
            pl.BlockSpec((BLK, D_),
                         lambda b, m: (jnp.minimum(b, m[OFF_NREAL] - 1), 0)),
            pl.BlockSpec(memory_space=pl.ANY),
            pl.BlockSpec(memory_space=pl.ANY),
            pl.BlockSpec((1, D_), lambda b, m: (0, 0)),
            pl.BlockSpec((1, D_), lambda b, m: (0, 0)),
        ],
        out_specs=pl.BlockSpec(
            (BLK, D_), lambda b, m: (jnp.minimum(b, m[OFF_NREAL] - 1), 0)),
        scratch_shapes=[
            pltpu.VMEM((2, D_, H_), jnp.float32),
            pltpu.VMEM((2, H_, D_), jnp.float32),
            pltpu.SemaphoreType.DMA((2,)),
        ],
    ),
    out_shape=jax.ShapeDtypeStruct((NPAD, D_), jnp.float32),
    compiler_params=pltpu.CompilerParams(
        dimension_semantics=("arbitrary",),
    ),
)


@functools.cache
def _sc_kernels():
    mesh = plsc.VectorSubcoreMesh(core_axis_name="c", subcore_axis_name="s")
    scratch = [
        pltpu.VMEM((SC_CH,), jnp.int32),
        pltpu.VMEM((SC_CH, D_), jnp.float32),
        pltpu.SemaphoreType.DMA,
    ]

    @functools.partial(
        pl.kernel,
        mesh=mesh,
        out_type=jax.ShapeDtypeStruct((NPAD, D_), jnp.float32),
        scratch_types=scratch,
    )
    def sc_scatter(x_hbm, pos_hbm, xs_hbm, idx_v, rows_v, sem):
        wid = lax.axis_index("s") * SC_NC + lax.axis_index("c")
        for ch in range(SC_NCH):
            base = wid * TOK_PER_W + ch * SC_CH
            pltpu.sync_copy(pos_hbm.at[pl.ds(base, SC_CH)], idx_v)
            pltpu.sync_copy(x_hbm.at[pl.ds(base, SC_CH)], rows_v)
            pltpu.async_copy(rows_v, xs_hbm.at[idx_v], sem).wait()

    @functools.partial(
        pl.kernel,
        mesh=mesh,
        out_type=jax.ShapeDtypeStruct((N_, D_), jnp.float32),
        scratch_types=scratch,
    )
    def sc_gather(ys_hbm, pos_hbm, out_hbm, idx_v, rows_v, sem):
        wid = lax.axis_index("s") * SC_NC + lax.axis_index("c")
        for ch in range(SC_NCH):
            base = wid * TOK_PER_W + ch * SC_CH
            pltpu.sync_copy(pos_hbm.at[pl.ds(base, SC_CH)], idx_v)
            pltpu.async_copy(ys_hbm.at[idx_v], rows_v, sem).wait()
            pltpu.sync_copy(rows_v, out_hbm.at[pl.ds(base, SC_CH)])

    return sc_scatter, sc_gather


def kernel(x, gate_w, expert_w1, expert_w2, ln_gamma, ln_beta):
    x_flat = x.reshape(N_, D_)
    pos8, be8 = _routing(x_flat, gate_w)
    pos = pos8[:, 0]
    be = be8[:, 0]
    sc_scatter, sc_gather = _sc_kernels()
    xs = sc_scatter(x_flat, pos)
    ys = _mlp(be, xs, expert_w1, expert_w2,
              ln_gamma.reshape(1, D_), ln_beta.reshape(1, D_))
    out = sc_gather(ys, pos)
    return out.reshape(B_, T_, D_)


# speedup vs baseline: 7.1507x; 1.0056x over previous
import functools

import jax
import jax.numpy as jnp
from jax import lax
from jax.experimental import pallas as pl
from jax.experimental.pallas import tpu as pltpu
from jax.experimental.pallas import tpu_sc as plsc

B_, T_, D_, H_, E_ = 2, 2048, 1024, 2048, 8
N_ = B_ * T_
BLK = 256
NBLK = (N_ + E_ * (BLK - 1) + BLK - 1) // BLK
NPAD = NBLK * BLK
CHUNK = 1024
OFF_SLOT = NBLK
OFF_BND = 2 * NBLK
OFF_NXE = 3 * NBLK
OFF_NREAL = 4 * NBLK
NMETA = 4 * NBLK + 8

SC_NC = 2
SC_NS = 16
SC_W = SC_NC * SC_NS
TOK_PER_W = N_ // SC_W
SC_CH = 32
SC_NCH = TOK_PER_W // SC_CH


def _routing_body(x_ref, gw_ref, pos_ref, be_ref):
    xv = x_ref[...]
    gw = gw_ref[...]
    logits = lax.dot_general(xv, gw, (((1,), (1,)), ((), ())),
                             preferred_element_type=jnp.float32)
    maxl = jnp.max(logits, axis=1, keepdims=True)
    p = jnp.exp(logits - maxl)
    p = p / jnp.sum(p, axis=1, keepdims=True)
    maxp = jnp.max(p, axis=1, keepdims=True)
    eids = lax.broadcasted_iota(jnp.int32, (N_, E_), 1)
    cand = jnp.where(p >= maxp, eids, E_)
    eidx = jnp.min(cand, axis=1, keepdims=True)
    onehot = (eids == eidx).astype(jnp.float32)

    r_i = lax.broadcasted_iota(jnp.int32, (CHUNK, CHUNK), 0)
    c_i = lax.broadcasted_iota(jnp.int32, (CHUNK, CHUNK), 1)
    tril = (r_i >= c_i).astype(jnp.bfloat16)
    base = jnp.zeros((1, E_), jnp.float32)
    cum_rows = []
    for c in range(N_ // CHUNK):
        oc = lax.slice(onehot, (c * CHUNK, 0), ((c + 1) * CHUNK, E_))
        local = jnp.dot(tril, oc.astype(jnp.bfloat16),
                        preferred_element_type=jnp.float32)
        cum_rows.append(local + base)
        base = base + jnp.sum(oc, axis=0, keepdims=True)
    cum = jnp.concatenate(cum_rows, axis=0)
    counts = base

    pc = jnp.ceil(counts / BLK) * BLK
    er = lax.broadcasted_iota(jnp.int32, (E_, E_), 0)
    ec = lax.broadcasted_iota(jnp.int32, (E_, E_), 1)
    sut = (er < ec).astype(jnp.float32)
    off8 = jnp.dot(jnp.broadcast_to(pc, (E_, E_)), sut,
                   preferred_element_type=jnp.float32)
    off = lax.slice(off8, (0, 0), (1, E_))

    rank = jnp.sum(cum * onehot, axis=1, keepdims=True) - 1.0
    poff = jnp.sum(off * onehot, axis=1, keepdims=True)
    pos = (poff + rank).astype(jnp.int32)
    pos_ref[...] = jnp.broadcast_to(pos, (N_, E_))

    bstart = (lax.broadcasted_iota(jnp.int32, (NBLK, E_), 0) * BLK
              ).astype(jnp.float32)
    beids = lax.broadcasted_iota(jnp.int32, (NBLK, E_), 1)
    offb = jnp.broadcast_to(off, (NBLK, E_))
    pcb = jnp.broadcast_to(pc, (NBLK, E_))
    ok = (offb <= bstart) & (pcb > 0)
    be = jnp.max(jnp.where(ok, beids, 0), axis=1, keepdims=True)
    beb = jnp.broadcast_to(be, (NBLK, E_))
    okn = (pcb > 0) & (beids > beb)
    nxt = jnp.min(jnp.where(okn, beids, E_), axis=1, keepdims=True)
    nxe = jnp.where(nxt == E_, be, nxt)
    okr = (pcb > 0) & (beids <= beb)
    rid = jnp.sum(okr.astype(jnp.int32), axis=1, keepdims=True) - 1
    slot = jnp.bitwise_and(rid, 1)
    be_prev = jnp.concatenate(
        [jnp.full((1, 1), -1, jnp.int32),
         lax.slice(be, (0, 0), (NBLK - 1, 1))], axis=0)
    bnd = (be != be_prev).astype(jnp.int32)
    nreal = (jnp.sum(pc, axis=1, keepdims=True) / BLK).astype(jnp.int32)
    meta = jnp.concatenate(
        [be, slot, bnd, nxe,
         jnp.broadcast_to(nreal, (NMETA - 4 * NBLK, 1))], axis=0)
    be_ref[...] = jnp.broadcast_to(meta, (NMETA, E_))


_routing = pl.pallas_call(
    _routing_body,
    out_shape=(
        jax.ShapeDtypeStruct((N_, E_), jnp.int32),
        jax.ShapeDtypeStruct((NMETA, E_), jnp.int32),
    ),
)


def _mlp_body(meta_ref, xs_ref, w1_hbm, w2_hbm, g_ref, b_ref, out_ref,
              w1buf, w2buf, sems):
    blk = pl.program_id(0)

    def w_copies(e, s):
        return (pltpu.make_async_copy(w1_hbm.at[e], w1buf.at[s], sems.at[s]),
                pltpu.make_async_copy(w2_hbm.at[e], w2buf.at[s], sems.at[s]))

    @pl.when(blk < meta_ref[OFF_NREAL])
    def _():
        e = meta_ref[blk]
        slot = meta_ref[OFF_SLOT + blk]
        first = meta_ref[OFF_BND + blk]
        nxe = meta_ref[OFF_NXE + blk]

        @pl.when(blk == 0)
        def _():
            for c in w_copies(e, slot):
                c.start()

        @pl.when(first == 1)
        def _():
            for c in w_copies(e, slot):
                c.wait()

            @pl.when(nxe != e)
            def _():
                for c in w_copies(nxe, 1 - slot):
                    c.start()

        hpre = jnp.dot(xs_ref[...].astype(jnp.bfloat16),
                       w1buf[slot].astype(jnp.bfloat16),
                       preferred_element_type=jnp.float32)
        hact = 0.5 * hpre * (1.0 + lax.erf(hpre * 0.7071067811865476))
        y = jnp.dot(hact.astype(jnp.bfloat16),
                    w2buf[slot].astype(jnp.bfloat16),
                    preferred_element_type=jnp.float32)
        mean = jnp.mean(y, axis=1, keepdims=True)
        yc = y - mean
        var = jnp.mean(yc * yc, axis=1, keepdims=True)
        out_ref[...] = yc * lax.rsqrt(var + 1e-5) * g_ref[...] + b_ref[...]


_mlp = pl.pallas_call(
    _mlp_body,
    grid_spec=pltpu.PrefetchScalarGridSpec(
        num_scalar_prefetch=1,
        grid=(NBLK,),
        in_specs=[
            pl.BlockSpec((BLK, D_),
                         lambda b, m: (jnp.minimum(b, m[OFF_NREAL] - 1), 0)),
            pl.BlockSpec(memory_space=pl.ANY),
            pl.BlockSpec(memory_space=pl.ANY),
            pl.BlockSpec((1, D_), lambda b, m: (0, 0)),
            pl.BlockSpec((1, D_), lambda b, m: (0, 0)),
        ],
        out_specs=pl.BlockSpec(
            (BLK, D_), lambda b, m: (jnp.minimum(b, m[OFF_NREAL] - 1), 0)),
        scratch_shapes=[
            pltpu.VMEM((2, D_, H_), jnp.float32),
            pltpu.VMEM((2, H_, D_), jnp.float32),
            pltpu.SemaphoreType.DMA((2,)),
        ],
    ),
    out_shape=jax.ShapeDtypeStruct((NPAD, D_), jnp.float32),
    compiler_params=pltpu.CompilerParams(
        dimension_semantics=("arbitrary",),
    ),
)


@functools.cache
def _sc_kernels():
    mesh = plsc.VectorSubcoreMesh(core_axis_name="c", subcore_axis_name="s")
    scratch = [
        pltpu.VMEM((2, SC_CH), jnp.int32),
        pltpu.VMEM((2, SC_CH, D_), jnp.float32),
        pltpu.SemaphoreType.DMA((2,)),
        pltpu.SemaphoreType.DMA((2,)),
    ]

    @functools.partial(
        pl.kernel,
        mesh=mesh,
        out_type=jax.ShapeDtypeStruct((NPAD, D_), jnp.float32),
        scratch_types=scratch,
    )
    def sc_scatter(x_hbm, pos_hbm, xs_hbm, idx_v, rows_v, rsem, wsem):
        wid = lax.axis_index("s") * SC_NC + lax.axis_index("c")
        base = wid * TOK_PER_W

        def rd(ch, s):
            return pltpu.make_async_copy(
                x_hbm.at[pl.ds(base + ch * SC_CH, SC_CH)],
                rows_v.at[s], rsem.at[s])

        def wr(s):
            return pltpu.make_async_copy(
                rows_v.at[s], xs_hbm.at[idx_v.at[s]], wsem.at[s])

        pltpu.sync_copy(pos_hbm.at[pl.ds(base, SC_CH)], idx_v.at[0])
        rd(0, 0).start()
        for ch in range(SC_NCH):
            s = ch % 2
            s2 = (ch + 1) % 2
            if ch + 1 < SC_NCH:
                if ch >= 1:
                    wr(s2).wait()
                pltpu.sync_copy(
                    pos_hbm.at[pl.ds(base + (ch + 1) * SC_CH, SC_CH)],
                    idx_v.at[s2])
                rd(ch + 1, s2).start()
            rd(ch, s).wait()
            wr(s).start()
        wr((SC_NCH - 2) % 2).wait()
        wr((SC_NCH - 1) % 2).wait()

    @functools.partial(
        pl.kernel,
        mesh=mesh,
        out_type=jax.ShapeDtypeStruct((N_, D_), jnp.float32),
        scratch_types=scratch,
    )
    def sc_gather(ys_hbm, pos_hbm, out_hbm, idx_v, rows_v, rsem, wsem):
        wid = lax.axis_index("s") * SC_NC + lax.axis_index("c")
        base = wid * TOK_PER_W

        def rd(s):
            return pltpu.make_async_copy(
                ys_hbm.at[idx_v.at[s]], rows_v.at[s], rsem.at[s])

        def wr(ch, s):
            return pltpu.make_async_copy(
                rows_v.at[s], out_hbm.at[pl.ds(base + ch * SC_CH, SC_CH)],
                wsem.at[s])

        pltpu.sync_copy(pos_hbm.at[pl.ds(base, SC_CH)], idx_v.at[0])
        rd(0).start()
        for ch in range(SC_NCH):
            s = ch % 2
            s2 = (ch + 1) % 2
            if ch + 1 < SC_NCH:
                if ch >= 1:
                    wr(ch - 1, s2).wait()
                pltpu.sync_copy(
                    pos_hbm.at[pl.ds(base + (ch + 1) * SC_CH, SC_CH)],
                    idx_v.at[s2])
                rd(s2).start()
            rd(s).wait()
            wr(ch, s).start()
        wr(SC_NCH - 2, (SC_NCH - 2) % 2).wait()
        wr(SC_NCH - 1, (SC_NCH - 1) % 2).wait()

    return sc_scatter, sc_gather


def kernel(x, gate_w, expert_w1, expert_w2, ln_gamma, ln_beta):
    x_flat = x.reshape(N_, D_)
    pos8, be8 = _routing(x_flat, gate_w)
    pos = pos8[:, 0]
    be = be8[:, 0]
    sc_scatter, sc_gather = _sc_kernels()
    xs = sc_scatter(x_flat, pos)
    ys = _mlp(be, xs, expert_w1, expert_w2,
              ln_gamma.reshape(1, D_), ln_beta.reshape(1, D_))
    out = sc_gather(ys, pos)
    return out.reshape(B_, T_, D_)
